# baseline (speedup 1.0000x reference)
import functools

import jax
import jax.numpy as jnp
from jax import lax
from jax.experimental import pallas as pl
from jax.experimental.pallas import tpu as pltpu
from jax.experimental.pallas import tpu_sc as plsc

N_NODES = 10000
N_CTX = 10000
N_EDGES = 320000
D = 128
L = 16

N_TILES = 16
CHUNK = 128
IDX_BLK = 16
N_BLKS = 10
STEPS = IDX_BLK * N_BLKS
EDGES_PER_TILE = CHUNK * STEPS
E_PAD = EDGES_PER_TILE * N_TILES
IDX_ROWS = E_PAD // CHUNK
ACC_ROWS = 10112
ROWS_PER_TILE = ACC_ROWS // N_TILES
FULL = ROWS_PER_TILE // CHUNK
REM = ROWS_PER_TILE % CHUNK
HIST = 10240

def _lane_masks():
    iota = lax.iota(jnp.int32, L)
    return [iota == i for i in range(L)]


def _sc_segment_sums(x2, src2, dst2, zrow):
    mesh = plsc.VectorSubcoreMesh(core_axis_name="c", subcore_axis_name="s")

    @functools.partial(
        pl.kernel,
        mesh=mesh,
        compiler_params=pltpu.CompilerParams(needs_layout_passes=False,
                                             use_tc_tiling_on_sc=False),
        out_type=[
            jax.ShapeDtypeStruct((2 * ACC_ROWS, D), jnp.float32),
            jax.ShapeDtypeStruct((2 * N_TILES, HIST), jnp.float32),
        ],
        scratch_types=[
            pltpu.VMEM((IDX_BLK, CHUNK), jnp.int32),
            pltpu.VMEM((IDX_BLK, CHUNK), jnp.int32),
            pltpu.VMEM((CHUNK, D // 2), jnp.int32),
            pltpu.VMEM((CHUNK, D // 2), jnp.int32),
            pltpu.VMEM((CHUNK, D), jnp.float32),
            pltpu.VMEM((HIST,), jnp.float32),
            pltpu.VMEM_SHARED((ACC_ROWS, D), jnp.float32),
            pltpu.SemaphoreType.DMA,
            pltpu.SemaphoreType.DMA,
            pltpu.SemaphoreType.DMA,
        ],
    )
    def k(x_h, src_h, dst_h, zrow_h, sum_h, cnt_h,
          src_t, dst_t, bf0, bf1, rows_f, hist_t, acc_s, sem0, sem1, sem2):
        g = lax.axis_index("c")
        s = lax.axis_index("s")
        row0 = s * ROWS_PER_TILE

        pltpu.sync_copy(zrow_h, rows_f)
        for kk in range(FULL):
            pltpu.sync_copy(rows_f, acc_s.at[pl.ds(row0 + kk * CHUNK, CHUNK)])
        pltpu.sync_copy(rows_f.at[pl.ds(0, REM)],
                        acc_s.at[pl.ds(row0 + FULL * CHUNK, REM)])

        zv = jnp.zeros((L,), jnp.float32)

        def zero_hist(i, carry):
            hist_t[pl.ds(i * L, L)] = zv
            return carry

        lax.fori_loop(0, HIST // L, zero_hist, 0)
        plsc.subcore_barrier()

        lanes = _lane_masks()
        onesv = jnp.ones((L,), jnp.float32)

        def count_row(j):
            def one(lane, carry):
                d = dst_t[j, pl.ds(lane * L, L)]
                for i in range(L):
                    plsc.addupdate_scatter(hist_t, [d], onesv, mask=lanes[i])
                return carry
            lax.fori_loop(0, CHUNK // L, one, 0)

        def wait_gather(j, rows, sem):
            pltpu.make_async_copy(x_h.at[src_t.at[j]], rows, sem).wait()

        def wait_scatter(j):
            pltpu.make_async_copy(rows_f, acc_s.at[dst_t.at[j]],
                                  sem2).wait()

        def convert(bf):
            def conv_row(r, carry):
                for c in range(D // 32):
                    w = bf[r, pl.ds(c * L, L)]
                    v = plsc.bitcast(w, jnp.bfloat16)
                    a, b = plsc.unpack(v, format=plsc.PackFormat.INTERLEAVED)
                    rows_f[r, pl.ds(c * 32, L)] = a
                    rows_f[r, pl.ds(c * 32 + L, L)] = b
                return carry

            lax.fori_loop(0, CHUNK, conv_row, 0)

        def outer(blk, carry):
            base = g * IDX_ROWS + s * STEPS + blk * IDX_BLK
            pltpu.sync_copy(src_h.at[pl.ds(base, IDX_BLK)], src_t)
            pltpu.sync_copy(dst_h.at[pl.ds(base, IDX_BLK)], dst_t)
            pltpu.async_copy(x_h.at[src_t.at[0]], bf0, sem0)

            def pair(u, carry2):
                ja = 2 * u
                jb = 2 * u + 1

                cpb = pltpu.async_copy(x_h.at[src_t.at[jb]], bf1, sem1)
                count_row(ja)
                wait_gather(ja, bf0, sem0)

                @pl.when(u > 0)
                def _():
                    wait_scatter(jb - 2)

                convert(bf0)
                pltpu.async_copy(rows_f, acc_s.at[dst_t.at[ja]], sem2,
                                 add=True)

                @pl.when(u < IDX_BLK // 2 - 1)
                def _():
                    pltpu.async_copy(x_h.at[src_t.at[ja + 2]], bf0, sem0)

                count_row(jb)
                cpb.wait()
                wait_scatter(ja)
                convert(bf1)
                pltpu.async_copy(rows_f, acc_s.at[dst_t.at[jb]], sem2,
                                 add=True)
                return carry2

            lax.fori_loop(0, IDX_BLK // 2, pair, carry)
            wait_scatter(IDX_BLK - 1)
            return carry

        lax.fori_loop(0, N_BLKS, outer, 0)
        plsc.subcore_barrier()

        out0 = g * ACC_ROWS + row0
        for kk in range(FULL):
            pltpu.sync_copy(acc_s.at[pl.ds(row0 + kk * CHUNK, CHUNK)], rows_f)
            pltpu.sync_copy(rows_f, sum_h.at[pl.ds(out0 + kk * CHUNK, CHUNK)])
        pltpu.sync_copy(acc_s.at[pl.ds(row0 + FULL * CHUNK, REM)],
                        rows_f.at[pl.ds(0, REM)])
        pltpu.sync_copy(rows_f.at[pl.ds(0, REM)],
                        sum_h.at[pl.ds(out0 + FULL * CHUNK, REM)])
        pltpu.sync_copy(hist_t, cnt_h.at[g * N_TILES + s])

    return k(x2, src2, dst2, zrow)


def _tc_body(sumA, cntA, sumB, cntB, ctx, wsrc, wdst2, b2, out):
    cA = jnp.maximum(jnp.sum(cntA[...], axis=1, keepdims=True), 1.0)
    cB = jnp.maximum(jnp.sum(cntB[...], axis=1, keepdims=True), 1.0)
    m = sumA[...] / cA + sumB[...] / cB
    acc = jnp.dot(m, wsrc[...], preferred_element_type=jnp.float32)
    acc += jnp.dot(ctx[...], wdst2[...], preferred_element_type=jnp.float32)
    out[...] = acc + b2[...]


def _tc_epilogue(sumA, cntA, sumB, cntB, context, W_src, W_dst2, b2):
    blk = 1000
    grid = (N_CTX // blk,)
    return pl.pallas_call(
        _tc_body,
        grid=grid,
        in_specs=[
            pl.BlockSpec((blk, D), lambda i: (i, 0)),
            pl.BlockSpec((blk, N_TILES), lambda i: (i, 0)),
            pl.BlockSpec((blk, D), lambda i: (i, 0)),
            pl.BlockSpec((blk, N_TILES), lambda i: (i, 0)),
            pl.BlockSpec((blk, D), lambda i: (i, 0)),
            pl.BlockSpec((D, D), lambda i: (0, 0)),
            pl.BlockSpec((D, D), lambda i: (0, 0)),
            pl.BlockSpec((1, D), lambda i: (0, 0)),
        ],
        out_specs=pl.BlockSpec((blk, D), lambda i: (i, 0)),
        out_shape=jax.ShapeDtypeStruct((N_CTX, D), jnp.float32),
    )(sumA, cntA, sumB, cntB, context, W_src, W_dst2, b2)


def _prep_edges(edges, src_off):
    src = edges[0].astype(jnp.int32) + src_off
    dst = edges[1].astype(jnp.int32)
    pad = E_PAD - N_EDGES
    src = jnp.concatenate([src, jnp.full((pad,), src_off, jnp.int32)])
    dst = jnp.concatenate([dst, jnp.full((pad,), N_CTX, jnp.int32)])
    return src.reshape(IDX_ROWS, CHUNK), dst.reshape(IDX_ROWS, CHUNK)


def kernel(xA, edgesA, xB, edgesB, context, W_src, W_dst, b):
    srcA, dstA = _prep_edges(edgesA, 0)
    srcB, dstB = _prep_edges(edgesB, N_NODES)
    x2 = jnp.concatenate([xA, xB])
    x2 = x2.reshape(-1, D // 32, 2, L).swapaxes(2, 3).reshape(-1, D)
    x2 = lax.bitcast_convert_type(
        x2.astype(jnp.bfloat16).reshape(-1, D // 2, 2), jnp.int32)
    src2 = jnp.concatenate([srcA, srcB])
    dst2 = jnp.concatenate([dstA, dstB])
    zrow = jnp.zeros((CHUNK, D), jnp.float32)
    sums, hists = _sc_segment_sums(x2, src2, dst2, zrow)
    cntA = hists[:N_TILES, :N_CTX].T
    cntB = hists[N_TILES:, :N_CTX].T
    return _tc_epilogue(sums[:N_CTX], cntA,
                        sums[ACC_ROWS:ACC_ROWS + N_CTX], cntB, context,
                        W_src, 2.0 * W_dst, (2.0 * b).reshape(1, D))


# speedup vs baseline: 1.2267x; 1.2267x over previous
import functools

import jax
import jax.numpy as jnp
from jax import lax
from jax.experimental import pallas as pl
from jax.experimental.pallas import tpu as pltpu
from jax.experimental.pallas import tpu_sc as plsc

N_NODES = 10000
N_CTX = 10000
N_EDGES = 320000
D = 128
L = 16

N_TILES = 16
CHUNK = 128
IDX_BLK = 16
N_BLKS = 10
STEPS = IDX_BLK * N_BLKS
EDGES_PER_TILE = CHUNK * STEPS
E_PAD = EDGES_PER_TILE * N_TILES
IDX_ROWS = E_PAD // CHUNK
ACC_ROWS = 10112
ROWS_PER_TILE = ACC_ROWS // N_TILES
FULL = ROWS_PER_TILE // CHUNK
REM = ROWS_PER_TILE % CHUNK
HIST = 10240

def _lane_masks():
    iota = lax.iota(jnp.int32, L)
    return [iota == i for i in range(L)]


def _sc_segment_sums(x2, src2, dst2, zrow):
    mesh = plsc.VectorSubcoreMesh(core_axis_name="c", subcore_axis_name="s")

    @functools.partial(
        pl.kernel,
        mesh=mesh,
        compiler_params=pltpu.CompilerParams(needs_layout_passes=False),
        out_type=[
            jax.ShapeDtypeStruct((2, ACC_ROWS, D), jnp.float32),
            jax.ShapeDtypeStruct((2, N_TILES, HIST), jnp.float32),
        ],
        scratch_types=[
            pltpu.VMEM((IDX_BLK, CHUNK), jnp.int32),
            pltpu.VMEM((IDX_BLK, CHUNK), jnp.int32),
            pltpu.VMEM((CHUNK, D), jnp.float32),
            pltpu.VMEM((CHUNK, D), jnp.float32),
            pltpu.VMEM((HIST,), jnp.float32),
            pltpu.VMEM_SHARED((ACC_ROWS, D), jnp.float32),
            pltpu.SemaphoreType.DMA,
            pltpu.SemaphoreType.DMA,
            pltpu.SemaphoreType.DMA,
            pltpu.SemaphoreType.DMA,
        ],
    )
    def k(x_h, src_h, dst_h, zrow_h, sum_h, cnt_h,
          src_t, dst_t, rows0, rows1, hist_t, acc_s, sem0, sem1, sem2, sem3):
        g = lax.axis_index("c")
        s = lax.axis_index("s")
        row0 = s * ROWS_PER_TILE

        pltpu.sync_copy(zrow_h, rows0)
        for kk in range(FULL):
            pltpu.sync_copy(rows0, acc_s.at[pl.ds(row0 + kk * CHUNK, CHUNK)])
        pltpu.sync_copy(rows0.at[pl.ds(0, REM)],
                        acc_s.at[pl.ds(row0 + FULL * CHUNK, REM)])

        zv = jnp.zeros((L,), jnp.float32)

        def zero_hist(i, carry):
            hist_t[pl.ds(i * L, L)] = zv
            return carry

        lax.fori_loop(0, HIST // L, zero_hist, 0)
        plsc.subcore_barrier()

        lanes = _lane_masks()
        onesv = jnp.ones((L,), jnp.float32)

        def count_row(j):
            def one(lane, carry):
                d = dst_t[j, pl.ds(lane * L, L)]
                for i in range(L):
                    plsc.addupdate_scatter(hist_t, [d], onesv, mask=lanes[i])
                return carry
            lax.fori_loop(0, CHUNK // L, one, 0)

        def wait_gather(j, rows, sem):
            pltpu.make_async_copy(x_h.at[src_t.at[j]], rows, sem).wait()

        def wait_scatter(j, rows, sem):
            pltpu.make_async_copy(rows, acc_s.at[dst_t.at[j]], sem).wait()

        def outer(blk, carry):
            base = g * IDX_ROWS + s * STEPS + blk * IDX_BLK
            pltpu.sync_copy(src_h.at[pl.ds(base, IDX_BLK)], src_t)
            pltpu.sync_copy(dst_h.at[pl.ds(base, IDX_BLK)], dst_t)
            pltpu.async_copy(x_h.at[src_t.at[0]], rows0, sem0)

            def pair(u, carry2):
                ja = 2 * u
                jb = 2 * u + 1

                @pl.when(u > 0)
                def _():
                    wait_scatter(jb - 2, rows1, sem3)

                cpb = pltpu.async_copy(x_h.at[src_t.at[jb]], rows1, sem1)
                count_row(ja)
                wait_gather(ja, rows0, sem0)
                pltpu.async_copy(rows0, acc_s.at[dst_t.at[ja]], sem2,
                                 add=True)
                count_row(jb)
                wait_scatter(ja, rows0, sem2)

                @pl.when(u < IDX_BLK // 2 - 1)
                def _():
                    pltpu.async_copy(x_h.at[src_t.at[ja + 2]], rows0, sem0)

                cpb.wait()
                pltpu.async_copy(rows1, acc_s.at[dst_t.at[jb]], sem3,
                                 add=True)
                return carry2

            lax.fori_loop(0, IDX_BLK // 2, pair, carry)
            wait_scatter(IDX_BLK - 1, rows1, sem3)
            return carry

        lax.fori_loop(0, N_BLKS, outer, 0)
        plsc.subcore_barrier()

        for kk in range(FULL):
            r = row0 + kk * CHUNK
            pltpu.sync_copy(acc_s.at[pl.ds(r, CHUNK)], rows0)
            pltpu.sync_copy(rows0, sum_h.at[g, pl.ds(r, CHUNK)])
        r = row0 + FULL * CHUNK
        pltpu.sync_copy(acc_s.at[pl.ds(r, REM)], rows0.at[pl.ds(0, REM)])
        pltpu.sync_copy(rows0.at[pl.ds(0, REM)], sum_h.at[g, pl.ds(r, REM)])
        pltpu.sync_copy(hist_t, cnt_h.at[g, s])

    return k(x2, src2, dst2, zrow)


def _tc_body(sums, cnts, ctx, wsrc, wdst2, b2, out):
    cA = jnp.maximum(jnp.sum(cnts[0], axis=1, keepdims=True), 1.0)
    cB = jnp.maximum(jnp.sum(cnts[1], axis=1, keepdims=True), 1.0)
    m = sums[0] / cA + sums[1] / cB
    acc = jnp.dot(m, wsrc[...], preferred_element_type=jnp.float32)
    acc += jnp.dot(ctx[...], wdst2[...], preferred_element_type=jnp.float32)
    out[...] = acc + b2[...]


def _tc_epilogue(sums, cnts_t, context, W_src, W_dst2, b2):
    blk = 1000
    grid = (N_CTX // blk,)
    return pl.pallas_call(
        _tc_body,
        grid=grid,
        in_specs=[
            pl.BlockSpec((2, blk, D), lambda i: (0, i, 0)),
            pl.BlockSpec((2, blk, N_TILES), lambda i: (0, i, 0)),
            pl.BlockSpec((blk, D), lambda i: (i, 0)),
            pl.BlockSpec((D, D), lambda i: (0, 0)),
            pl.BlockSpec((D, D), lambda i: (0, 0)),
            pl.BlockSpec((1, D), lambda i: (0, 0)),
        ],
        out_specs=pl.BlockSpec((blk, D), lambda i: (i, 0)),
        out_shape=jax.ShapeDtypeStruct((N_CTX, D), jnp.float32),
    )(sums, cnts_t, context, W_src, W_dst2, b2)


def _prep_edges(edges, src_off):
    src = edges[0].astype(jnp.int32) + src_off
    dst = edges[1].astype(jnp.int32)
    pad = E_PAD - N_EDGES
    src = jnp.concatenate([src, jnp.full((pad,), src_off, jnp.int32)])
    dst = jnp.concatenate([dst, jnp.full((pad,), N_CTX, jnp.int32)])
    return src.reshape(IDX_ROWS, CHUNK), dst.reshape(IDX_ROWS, CHUNK)


def kernel(xA, edgesA, xB, edgesB, context, W_src, W_dst, b):
    srcA, dstA = _prep_edges(edgesA, 0)
    srcB, dstB = _prep_edges(edgesB, N_NODES)
    x2 = jnp.concatenate([xA, xB])
    src2 = jnp.concatenate([srcA, srcB])
    dst2 = jnp.concatenate([dstA, dstB])
    zrow = jnp.zeros((CHUNK, D), jnp.float32)
    sums, hists = _sc_segment_sums(x2, src2, dst2, zrow)
    cnts_t = hists.transpose(0, 2, 1)
    return _tc_epilogue(sums, cnts_t, context,
                        W_src, 2.0 * W_dst, (2.0 * b).reshape(1, D))


# speedup vs baseline: 1.2721x; 1.0370x over previous
import functools

import jax
import jax.numpy as jnp
from jax import lax
from jax.experimental import pallas as pl
from jax.experimental.pallas import tpu as pltpu
from jax.experimental.pallas import tpu_sc as plsc

N_NODES = 10000
N_CTX = 10000
N_EDGES = 320000
D = 128
L = 16

N_TILES = 16
CHUNK = 128
IDX_BLK = 16
N_BLKS = 10
STEPS = IDX_BLK * N_BLKS
EDGES_PER_TILE = CHUNK * STEPS
E_PAD = EDGES_PER_TILE * N_TILES
IDX_ROWS = E_PAD // CHUNK
ACC_ROWS = 10112
ROWS_PER_TILE = ACC_ROWS // N_TILES
FULL = ROWS_PER_TILE // CHUNK
REM = ROWS_PER_TILE % CHUNK
HIST = 10240

def _lane_masks():
    iota = lax.iota(jnp.int32, L)
    return [iota == i for i in range(L)]


def _sc_segment_sums(x2, src2, dst2, zrow):
    mesh = plsc.VectorSubcoreMesh(core_axis_name="c", subcore_axis_name="s")

    @functools.partial(
        pl.kernel,
        mesh=mesh,
        compiler_params=pltpu.CompilerParams(needs_layout_passes=False),
        out_type=[
            jax.ShapeDtypeStruct((2, ACC_ROWS, D), jnp.float32),
            jax.ShapeDtypeStruct((2, N_TILES, HIST), jnp.float32),
        ],
        scratch_types=[
            pltpu.VMEM((IDX_BLK, CHUNK), jnp.int32),
            pltpu.VMEM((IDX_BLK, CHUNK), jnp.int32),
            pltpu.VMEM((1, CHUNK), jnp.int32),
            pltpu.VMEM((CHUNK, D), jnp.float32),
            pltpu.VMEM((CHUNK, D), jnp.float32),
            pltpu.VMEM((HIST,), jnp.float32),
            pltpu.VMEM_SHARED((ACC_ROWS, D), jnp.float32),
            pltpu.SemaphoreType.DMA,
            pltpu.SemaphoreType.DMA,
            pltpu.SemaphoreType.DMA,
            pltpu.SemaphoreType.DMA,
        ],
    )
    def k(x_h, src_h, dst_h, zrow_h, sum_h, cnt_h,
          src_t, dst_t, srcn_t, rows0, rows1, hist_t, acc_s,
          sem0, sem1, sem2, sem3):
        g = lax.axis_index("c")
        s = lax.axis_index("s")
        row0 = s * ROWS_PER_TILE

        pltpu.sync_copy(zrow_h, rows0)
        for kk in range(FULL):
            pltpu.sync_copy(rows0, acc_s.at[pl.ds(row0 + kk * CHUNK, CHUNK)])
        pltpu.sync_copy(rows0.at[pl.ds(0, REM)],
                        acc_s.at[pl.ds(row0 + FULL * CHUNK, REM)])

        zv = jnp.zeros((L,), jnp.float32)

        def zero_hist(i, carry):
            hist_t[pl.ds(i * L, L)] = zv
            return carry

        lax.fori_loop(0, HIST // L, zero_hist, 0)
        plsc.subcore_barrier()

        lanes = _lane_masks()
        onesv = jnp.ones((L,), jnp.float32)

        def count_row(j):
            def one(lane, carry):
                d = dst_t[j, pl.ds(lane * L, L)]
                for i in range(L):
                    plsc.addupdate_scatter(hist_t, [d], onesv, mask=lanes[i])
                return carry
            lax.fori_loop(0, CHUNK // L, one, 0)

        def wait_gather(j, rows, sem):
            pltpu.make_async_copy(x_h.at[src_t.at[j]], rows, sem).wait()

        def wait_scatter(j, rows, sem):
            pltpu.make_async_copy(rows, acc_s.at[dst_t.at[j]], sem).wait()

        base0 = g * IDX_ROWS + s * STEPS
        pltpu.sync_copy(src_h.at[pl.ds(base0, 1)], srcn_t)
        pltpu.async_copy(x_h.at[srcn_t.at[0]], rows0, sem0)

        def outer(blk, carry):
            base = base0 + blk * IDX_BLK
            pltpu.sync_copy(src_h.at[pl.ds(base, IDX_BLK)], src_t)
            pltpu.sync_copy(dst_h.at[pl.ds(base, IDX_BLK)], dst_t)

            def pair(u, carry2):
                ja = 2 * u
                jb = 2 * u + 1

                @pl.when(u > 0)
                def _():
                    wait_scatter(jb - 2, rows1, sem3)

                cpb = pltpu.async_copy(x_h.at[src_t.at[jb]], rows1, sem1)
                count_row(ja)
                wait_gather(ja, rows0, sem0)

                @pl.when((u == 0) & (blk < N_BLKS - 1))
                def _():
                    pltpu.sync_copy(src_h.at[pl.ds(base + IDX_BLK, 1)],
                                    srcn_t)

                pltpu.async_copy(rows0, acc_s.at[dst_t.at[ja]], sem2,
                                 add=True)
                count_row(jb)
                wait_scatter(ja, rows0, sem2)

                @pl.when(u < IDX_BLK // 2 - 1)
                def _():
                    pltpu.async_copy(x_h.at[src_t.at[ja + 2]], rows0, sem0)

                @pl.when((u == IDX_BLK // 2 - 1) & (blk < N_BLKS - 1))
                def _():
                    pltpu.async_copy(x_h.at[srcn_t.at[0]], rows0, sem0)

                cpb.wait()
                pltpu.async_copy(rows1, acc_s.at[dst_t.at[jb]], sem3,
                                 add=True)
                return carry2

            lax.fori_loop(0, IDX_BLK // 2, pair, carry)
            wait_scatter(IDX_BLK - 1, rows1, sem3)
            return carry

        lax.fori_loop(0, N_BLKS, outer, 0)
        plsc.subcore_barrier()

        for kk in range(FULL):
            r = row0 + kk * CHUNK
            pltpu.sync_copy(acc_s.at[pl.ds(r, CHUNK)], rows0)
            pltpu.sync_copy(rows0, sum_h.at[g, pl.ds(r, CHUNK)])
        r = row0 + FULL * CHUNK
        pltpu.sync_copy(acc_s.at[pl.ds(r, REM)], rows0.at[pl.ds(0, REM)])
        pltpu.sync_copy(rows0.at[pl.ds(0, REM)], sum_h.at[g, pl.ds(r, REM)])
        pltpu.sync_copy(hist_t, cnt_h.at[g, s])

    return k(x2, src2, dst2, zrow)


def _tc_body(sums, cnts, ctx, wsrc, wdst2, b2, out):
    cA = jnp.maximum(jnp.sum(cnts[0], axis=1, keepdims=True), 1.0)
    cB = jnp.maximum(jnp.sum(cnts[1], axis=1, keepdims=True), 1.0)
    m = sums[0] / cA + sums[1] / cB
    acc = jnp.dot(m, wsrc[...], preferred_element_type=jnp.float32)
    acc += jnp.dot(ctx[...], wdst2[...], preferred_element_type=jnp.float32)
    out[...] = acc + b2[...]


def _tc_epilogue(sums, cnts_t, context, W_src, W_dst2, b2):
    blk = 1000
    grid = (N_CTX // blk,)
    return pl.pallas_call(
        _tc_body,
        grid=grid,
        in_specs=[
            pl.BlockSpec((2, blk, D), lambda i: (0, i, 0)),
            pl.BlockSpec((2, blk, N_TILES), lambda i: (0, i, 0)),
            pl.BlockSpec((blk, D), lambda i: (i, 0)),
            pl.BlockSpec((D, D), lambda i: (0, 0)),
            pl.BlockSpec((D, D), lambda i: (0, 0)),
            pl.BlockSpec((1, D), lambda i: (0, 0)),
        ],
        out_specs=pl.BlockSpec((blk, D), lambda i: (i, 0)),
        out_shape=jax.ShapeDtypeStruct((N_CTX, D), jnp.float32),
    )(sums, cnts_t, context, W_src, W_dst2, b2)


def _prep_edges(edges, src_off):
    src = edges[0].astype(jnp.int32) + src_off
    dst = edges[1].astype(jnp.int32)
    pad = E_PAD - N_EDGES
    src = jnp.concatenate([src, jnp.full((pad,), src_off, jnp.int32)])
    dst = jnp.concatenate([dst, jnp.full((pad,), N_CTX, jnp.int32)])
    return src.reshape(IDX_ROWS, CHUNK), dst.reshape(IDX_ROWS, CHUNK)


def kernel(xA, edgesA, xB, edgesB, context, W_src, W_dst, b):
    srcA, dstA = _prep_edges(edgesA, 0)
    srcB, dstB = _prep_edges(edgesB, N_NODES)
    x2 = jnp.concatenate([xA, xB])
    src2 = jnp.concatenate([srcA, srcB])
    dst2 = jnp.concatenate([dstA, dstB])
    zrow = jnp.zeros((CHUNK, D), jnp.float32)
    sums, hists = _sc_segment_sums(x2, src2, dst2, zrow)
    cnts_t = hists.transpose(0, 2, 1)
    return _tc_epilogue(sums, cnts_t, context,
                        W_src, 2.0 * W_dst, (2.0 * b).reshape(1, D))
